# baseline (speedup 1.0000x reference)
import functools

import jax
import jax.numpy as jnp
from jax import lax
from jax.experimental import pallas as pl
from jax.experimental.pallas import tpu as pltpu
from jax.experimental.pallas import tpu_sc as plsc

VOCAB_DIM = 64
SCALE = 8.0

_info = plsc.get_sparse_core_info()
NC, NS, L = _info.num_cores, _info.num_subcores, _info.num_lanes
NW = NC * NS

BATCH = 128


def _emb_body(n_batches_per_w, table_hbm, idx_hbm, out_hbm, idx_v, rows_v, sem):
    wid = lax.axis_index("s") * NC + lax.axis_index("c")
    base_b = wid * n_batches_per_w
    pltpu.sync_copy(idx_hbm.at[pl.ds(base_b, n_batches_per_w)], idx_v)

    def batch_body(b, carry):
        pltpu.async_copy(table_hbm.at[idx_v.at[b]], rows_v, sem).wait()

        def row_body(r, c):
            for j in range(VOCAB_DIM // 16):
                rows_v[r, pl.ds(j * 16, 16)] = rows_v[r, pl.ds(j * 16, 16)] * SCALE
            return c

        lax.fori_loop(0, BATCH, row_body, 0, unroll=4)
        pltpu.sync_copy(rows_v, out_hbm.at[pl.ds((base_b + b) * BATCH, BATCH)])
        return carry

    lax.fori_loop(0, n_batches_per_w, batch_body, 0)


def kernel(x, table):
    n_rows = x.shape[0] * x.shape[1]
    assert n_rows % (NW * BATCH) == 0
    n_batches = n_rows // BATCH
    n_batches_per_w = n_batches // NW

    idx = x.reshape(n_batches, BATCH).astype(jnp.int32)

    mesh = plsc.VectorSubcoreMesh(core_axis_name="c", subcore_axis_name="s")
    k = pl.kernel(
        functools.partial(_emb_body, n_batches_per_w),
        mesh=mesh,
        out_type=jax.ShapeDtypeStruct((n_rows, VOCAB_DIM), jnp.float32),
        scratch_types=[
            pltpu.VMEM((n_batches_per_w, BATCH), jnp.int32),
            pltpu.VMEM((BATCH, VOCAB_DIM), jnp.float32),
            pltpu.SemaphoreType.DMA,
        ],
        compiler_params=pltpu.CompilerParams(use_tc_tiling_on_sc=False),
    )
    out = k(table, idx)
    return out.reshape(x.shape[0], x.shape[1], VOCAB_DIM)
